# baseline (device time: 48555 ns/iter reference)
import jax
import jax.numpy as jnp
from jax import lax
from jax.experimental import pallas as pl
from jax.experimental.pallas import tpu as pltpu

N_DEV = 4
SCALE = 0.08838834764831843


def kernel(x, Wq, Wo, K_ext, V_ext):
    _, Sq, D = x.shape
    _, Skv, Hq, Dh = K_ext.shape

    def body(x_ref, wq_ref, wo_ref, k_ref, v_ref, out_ref,
             attn_ref, sbuf_ref, comm_ref, kbuf, vbuf,
             ksems, vsems, send_sems, recv_sems):
        my = lax.axis_index("i")
        left = lax.rem(my + N_DEV - 1, N_DEV)
        right = lax.rem(my + 1, N_DEV)

        barrier = pltpu.get_barrier_semaphore()
        pl.semaphore_signal(barrier, inc=1, device_id=(left,),
                            device_id_type=pl.DeviceIdType.MESH)
        pl.semaphore_signal(barrier, inc=1, device_id=(right,),
                            device_id_type=pl.DeviceIdType.MESH)
        pl.semaphore_wait(barrier, 2)

        def start_fetch(h):
            slot = h % 2
            ck = pltpu.make_async_copy(
                k_ref.at[0, :, h, :], kbuf.at[slot], ksems.at[slot])
            cv = pltpu.make_async_copy(
                v_ref.at[0, :, h, :], vbuf.at[slot], vsems.at[slot])
            ck.start()
            cv.start()
            return ck, cv

        fetches = {0: start_fetch(0)}

        xb = x_ref[...].astype(jnp.bfloat16)
        wqb = wq_ref[...].astype(jnp.bfloat16)
        q = lax.dot_general(xb, wqb, (((1,), (0,)), ((), ())),
                            preferred_element_type=jnp.float32)

        for h in range(Hq):
            if h + 1 < Hq:
                fetches[h + 1] = start_fetch(h + 1)
            ck, cv = fetches.pop(h)
            ck.wait()
            cv.wait()
            slot = h % 2
            qh = (q[:, h * Dh:(h + 1) * Dh] * SCALE).astype(jnp.bfloat16)
            qht = jnp.transpose(qh)
            kh = kbuf[slot].astype(jnp.bfloat16)
            st = lax.dot_general(kh, qht, (((1,), (0,)), ((), ())),
                                 preferred_element_type=jnp.float32)
            pt = jnp.exp(st).astype(jnp.bfloat16)
            ones = jnp.ones((Skv, 1), dtype=jnp.bfloat16)
            l = lax.dot_general(ones, pt, (((0,), (0,)), ((), ())),
                                preferred_element_type=jnp.float32)
            vh = vbuf[slot].astype(jnp.bfloat16)
            o = lax.dot_general(pt, vh,
                                (((0,), (0,)), ((), ())),
                                preferred_element_type=jnp.float32)
            attn_ref[:, h * Dh:(h + 1) * Dh] = (
                o / jnp.transpose(l)).astype(jnp.bfloat16)

        wob = wo_ref[...].astype(jnp.bfloat16)
        part = lax.dot_general(attn_ref[...], wob, (((1,), (0,)), ((), ())),
                               preferred_element_type=jnp.float32)
        out_ref[0] = part

        for phase in range(2):
            partner = my ^ (1 << phase)
            sbuf_ref[phase] = out_ref[0].astype(jnp.bfloat16)
            rdma = pltpu.make_async_remote_copy(
                src_ref=sbuf_ref.at[phase],
                dst_ref=comm_ref.at[phase],
                send_sem=send_sems.at[phase],
                recv_sem=recv_sems.at[phase],
                device_id=(partner,),
                device_id_type=pl.DeviceIdType.MESH,
            )
            rdma.start()
            rdma.wait()
            out_ref[0] = out_ref[0] + comm_ref[phase].astype(jnp.float32)

    out = pl.pallas_call(
        body,
        out_shape=jax.ShapeDtypeStruct((1, Sq, D), jnp.float32),
        in_specs=[
            pl.BlockSpec(memory_space=pltpu.VMEM),
            pl.BlockSpec(memory_space=pltpu.VMEM),
            pl.BlockSpec(memory_space=pltpu.VMEM),
            pl.BlockSpec(memory_space=pl.ANY),
            pl.BlockSpec(memory_space=pl.ANY),
        ],
        out_specs=pl.BlockSpec(memory_space=pltpu.VMEM),
        scratch_shapes=[
            pltpu.VMEM((Sq, Hq * Dh), jnp.bfloat16),
            pltpu.VMEM((2, Sq, D), jnp.bfloat16),
            pltpu.VMEM((2, Sq, D), jnp.bfloat16),
            pltpu.VMEM((2, Skv, Dh), jnp.float32),
            pltpu.VMEM((2, Skv, Dh), jnp.float32),
            pltpu.SemaphoreType.DMA((2,)),
            pltpu.SemaphoreType.DMA((2,)),
            pltpu.SemaphoreType.DMA((2,)),
            pltpu.SemaphoreType.DMA((2,)),
        ],
        compiler_params=pltpu.CompilerParams(
            collective_id=0, vmem_limit_bytes=100 * 1024 * 1024),
    )(x.reshape(Sq, D), Wq, Wo, K_ext, V_ext)
    return out


# device time: 48236 ns/iter; 1.0066x vs baseline; 1.0066x over previous
import jax
import jax.numpy as jnp
from jax import lax
from jax.experimental import pallas as pl
from jax.experimental.pallas import tpu as pltpu

N_DEV = 4
SCALE = 0.08838834764831843
NB = 2
BR = 128


def kernel(x, Wq, Wo, K_ext, V_ext):
    _, Sq, D = x.shape
    _, Skv, Hq, Dh = K_ext.shape

    def body(x_ref, wq_ref, wo_ref, k_ref, v_ref, out_ref,
             attn_ref, sbuf_ref, comm_ref, kbuf, vbuf,
             ksems, vsems, send_sems, recv_sems):
        my = lax.axis_index("i")
        left = lax.rem(my + N_DEV - 1, N_DEV)
        right = lax.rem(my + 1, N_DEV)

        barrier = pltpu.get_barrier_semaphore()
        pl.semaphore_signal(barrier, inc=1, device_id=(left,),
                            device_id_type=pl.DeviceIdType.MESH)
        pl.semaphore_signal(barrier, inc=1, device_id=(right,),
                            device_id_type=pl.DeviceIdType.MESH)
        pl.semaphore_wait(barrier, 2)

        def start_fetch(i):
            h = i % Hq
            slot = i % 2
            ck = pltpu.make_async_copy(
                k_ref.at[0, :, h, :], kbuf.at[slot], ksems.at[slot])
            cv = pltpu.make_async_copy(
                v_ref.at[0, :, h, :], vbuf.at[slot], vsems.at[slot])
            ck.start()
            cv.start()
            return ck, cv

        fetches = {0: start_fetch(0)}

        def launch(b, p):
            partner = my ^ (1 << p)
            r = pltpu.make_async_remote_copy(
                src_ref=sbuf_ref.at[p, b * BR:(b + 1) * BR],
                dst_ref=comm_ref.at[p, b * BR:(b + 1) * BR],
                send_sem=send_sems.at[p, b],
                recv_sem=recv_sems.at[p, b],
                device_id=(partner,),
                device_id_type=pl.DeviceIdType.MESH,
            )
            r.start()
            return r

        def accumulate(b, p):
            rows = slice(b * BR, (b + 1) * BR)
            acc = out_ref[0, rows] + comm_ref[p, rows].astype(jnp.float32)
            out_ref[0, rows] = acc
            return acc

        comms = {}

        xb = x_ref[...].astype(jnp.bfloat16)
        wqb = wq_ref[...].astype(jnp.bfloat16)
        q = lax.dot_general(xb, wqb, (((1,), (0,)), ((), ())),
                            preferred_element_type=jnp.float32)
        wob = wo_ref[...].astype(jnp.bfloat16)

        for b in range(NB):
            rows = slice(b * BR, (b + 1) * BR)
            for h in range(Hq):
                i = b * Hq + h
                if i + 1 < NB * Hq:
                    fetches[i + 1] = start_fetch(i + 1)
                ck, cv = fetches.pop(i)
                ck.wait()
                cv.wait()
                slot = i % 2

                if b == 1 and h == 3:
                    comms.pop((0, 0)).wait()
                    acc = accumulate(0, 0)
                    sbuf_ref[1, 0:BR] = acc.astype(jnp.bfloat16)
                    comms[(0, 1)] = launch(0, 1)

                qh = (q[rows, h * Dh:(h + 1) * Dh] * SCALE).astype(jnp.bfloat16)
                qht = jnp.transpose(qh)
                kh = kbuf[slot].astype(jnp.bfloat16)
                st = lax.dot_general(kh, qht, (((1,), (0,)), ((), ())),
                                     preferred_element_type=jnp.float32)
                pt = jnp.exp(st)
                l = jnp.sum(pt, axis=0, keepdims=True)
                vh = vbuf[slot].astype(jnp.bfloat16)
                o = lax.dot_general(pt.astype(jnp.bfloat16), vh,
                                    (((0,), (0,)), ((), ())),
                                    preferred_element_type=jnp.float32)
                attn_ref[rows, h * Dh:(h + 1) * Dh] = (
                    o / jnp.transpose(l)).astype(jnp.bfloat16)

            part = lax.dot_general(attn_ref[rows, :], wob,
                                   (((1,), (0,)), ((), ())),
                                   preferred_element_type=jnp.float32)
            out_ref[0, rows] = part
            sbuf_ref[0, rows] = part.astype(jnp.bfloat16)
            comms[(b, 0)] = launch(b, 0)

        comms.pop((0, 1)).wait()
        accumulate(0, 1)

        comms.pop((1, 0)).wait()
        acc = accumulate(1, 0)
        sbuf_ref[1, BR:2 * BR] = acc.astype(jnp.bfloat16)
        r = launch(1, 1)
        r.wait()
        accumulate(1, 1)

    out = pl.pallas_call(
        body,
        out_shape=jax.ShapeDtypeStruct((1, Sq, D), jnp.float32),
        in_specs=[
            pl.BlockSpec(memory_space=pltpu.VMEM),
            pl.BlockSpec(memory_space=pltpu.VMEM),
            pl.BlockSpec(memory_space=pltpu.VMEM),
            pl.BlockSpec(memory_space=pl.ANY),
            pl.BlockSpec(memory_space=pl.ANY),
        ],
        out_specs=pl.BlockSpec(memory_space=pltpu.VMEM),
        scratch_shapes=[
            pltpu.VMEM((Sq, Hq * Dh), jnp.bfloat16),
            pltpu.VMEM((2, Sq, D), jnp.bfloat16),
            pltpu.VMEM((2, Sq, D), jnp.bfloat16),
            pltpu.VMEM((2, Skv, Dh), jnp.float32),
            pltpu.VMEM((2, Skv, Dh), jnp.float32),
            pltpu.SemaphoreType.DMA((2,)),
            pltpu.SemaphoreType.DMA((2,)),
            pltpu.SemaphoreType.DMA((2, NB)),
            pltpu.SemaphoreType.DMA((2, NB)),
        ],
        compiler_params=pltpu.CompilerParams(
            collective_id=0, vmem_limit_bytes=100 * 1024 * 1024),
    )(x.reshape(Sq, D), Wq, Wo, K_ext, V_ext)
    return out


# device time: 43991 ns/iter; 1.1037x vs baseline; 1.0965x over previous
import jax
import jax.numpy as jnp
from jax import lax
from jax.experimental import pallas as pl
from jax.experimental.pallas import tpu as pltpu

N_DEV = 4
SCALE = 0.08838834764831843


def kernel(x, Wq, Wo, K_ext, V_ext):
    _, Sq, D = x.shape
    _, Skv, Hq, Dh = K_ext.shape

    def body(x_ref, wq_ref, wo_ref, k_ref, v_ref, out_ref,
             attn_ref, sbuf_ref, comm_ref, kbuf, vbuf,
             ksems, vsems, send_sems, recv_sems):
        my = lax.axis_index("i")
        left = lax.rem(my + N_DEV - 1, N_DEV)
        right = lax.rem(my + 1, N_DEV)

        barrier = pltpu.get_barrier_semaphore()
        pl.semaphore_signal(barrier, inc=1, device_id=(left,),
                            device_id_type=pl.DeviceIdType.MESH)
        pl.semaphore_signal(barrier, inc=1, device_id=(right,),
                            device_id_type=pl.DeviceIdType.MESH)
        pl.semaphore_wait(barrier, 2)

        def start_fetch(h):
            slot = h % 2
            ck = pltpu.make_async_copy(
                k_ref.at[0, :, h, :], kbuf.at[slot], ksems.at[slot])
            cv = pltpu.make_async_copy(
                v_ref.at[0, :, h, :], vbuf.at[slot], vsems.at[slot])
            ck.start()
            cv.start()
            return ck, cv

        fetches = {0: start_fetch(0)}

        xb = x_ref[...].astype(jnp.bfloat16)
        wqb = wq_ref[...].astype(jnp.bfloat16)
        q = lax.dot_general(xb, wqb, (((1,), (0,)), ((), ())),
                            preferred_element_type=jnp.float32)

        for h in range(Hq):
            if h + 1 < Hq:
                fetches[h + 1] = start_fetch(h + 1)
            ck, cv = fetches.pop(h)
            ck.wait()
            cv.wait()
            slot = h % 2
            qh = (q[:, h * Dh:(h + 1) * Dh] * SCALE).astype(jnp.bfloat16)
            qht = jnp.transpose(qh)
            kh = kbuf[slot].astype(jnp.bfloat16)
            st = lax.dot_general(kh, qht, (((1,), (0,)), ((), ())),
                                 preferred_element_type=jnp.float32)
            pt = jnp.exp(st)
            l = jnp.sum(pt, axis=0, keepdims=True)
            vh = vbuf[slot].astype(jnp.bfloat16)
            o = lax.dot_general(pt.astype(jnp.bfloat16), vh,
                                (((0,), (0,)), ((), ())),
                                preferred_element_type=jnp.float32)
            attn_ref[:, h * Dh:(h + 1) * Dh] = (
                o / jnp.transpose(l)).astype(jnp.bfloat16)

        wob = wo_ref[...].astype(jnp.bfloat16)
        part = lax.dot_general(attn_ref[...], wob, (((1,), (0,)), ((), ())),
                               preferred_element_type=jnp.float32)
        out_ref[0] = part

        for phase in range(2):
            partner = my ^ (1 << phase)
            sbuf_ref[phase] = out_ref[0].astype(jnp.bfloat16)
            rdma = pltpu.make_async_remote_copy(
                src_ref=sbuf_ref.at[phase],
                dst_ref=comm_ref.at[phase],
                send_sem=send_sems.at[phase],
                recv_sem=recv_sems.at[phase],
                device_id=(partner,),
                device_id_type=pl.DeviceIdType.MESH,
            )
            rdma.start()
            rdma.wait()
            out_ref[0] = out_ref[0] + comm_ref[phase].astype(jnp.float32)

    out = pl.pallas_call(
        body,
        out_shape=jax.ShapeDtypeStruct((1, Sq, D), jnp.float32),
        in_specs=[
            pl.BlockSpec(memory_space=pltpu.VMEM),
            pl.BlockSpec(memory_space=pltpu.VMEM),
            pl.BlockSpec(memory_space=pltpu.VMEM),
            pl.BlockSpec(memory_space=pl.ANY),
            pl.BlockSpec(memory_space=pl.ANY),
        ],
        out_specs=pl.BlockSpec(memory_space=pltpu.VMEM),
        scratch_shapes=[
            pltpu.VMEM((Sq, Hq * Dh), jnp.bfloat16),
            pltpu.VMEM((2, Sq, D), jnp.bfloat16),
            pltpu.VMEM((2, Sq, D), jnp.bfloat16),
            pltpu.VMEM((2, Skv, Dh), jnp.float32),
            pltpu.VMEM((2, Skv, Dh), jnp.float32),
            pltpu.SemaphoreType.DMA((2,)),
            pltpu.SemaphoreType.DMA((2,)),
            pltpu.SemaphoreType.DMA((2,)),
            pltpu.SemaphoreType.DMA((2,)),
        ],
        compiler_params=pltpu.CompilerParams(
            collective_id=0, vmem_limit_bytes=100 * 1024 * 1024),
    )(x.reshape(Sq, D), Wq, Wo, K_ext, V_ext)
    return out
